# baseline (device time: 28726 ns/iter reference)
import jax
import jax.numpy as jnp
from jax import lax
from jax.experimental import pallas as pl
from jax.experimental.pallas import tpu as pltpu

N_DEV = 4


def kernel(x, router_W, route_idx, expert_W):
    n_tok, d = x.shape
    e_loc, _, h_dim = expert_W.shape
    n_exp = N_DEV * e_loc
    e_half = e_loc // 2

    def body(x_ref, rw_ref, idx_ref, ew_ref, out_ref,
             my_w, w_fL, w_fR, w_diag, send_sems, recv_sems):
        my = lax.axis_index("i")
        left = lax.rem(my + N_DEV - 1, N_DEV)
        right = lax.rem(my + 1, N_DEV)
        diag = lax.rem(my + 2, N_DEV)

        lo = pl.ds(0, e_half)
        hi = pl.ds(e_half, e_half)

        barrier_sem = pltpu.get_barrier_semaphore()
        for nbr in (left, right):
            pl.semaphore_signal(
                barrier_sem, inc=1,
                device_id=(nbr,), device_id_type=pl.DeviceIdType.MESH,
            )

        my_w[...] = ew_ref[...].astype(jnp.bfloat16)

        pl.semaphore_wait(barrier_sem, 2)

        def copy(src, dst, s_sem, r_sem, dev):
            return pltpu.make_async_remote_copy(
                src_ref=src, dst_ref=dst,
                send_sem=send_sems.at[s_sem], recv_sem=recv_sems.at[r_sem],
                device_id=(dev,), device_id_type=pl.DeviceIdType.MESH,
            )

        sR0 = copy(my_w.at[lo], w_fL.at[lo], 0, 0, right)
        sL0 = copy(my_w.at[hi], w_fR.at[hi], 2, 2, left)
        sR1 = copy(my_w.at[hi], w_fL.at[hi], 1, 1, right)
        sL1 = copy(my_w.at[lo], w_fR.at[lo], 3, 3, left)
        sR0.start()
        sL0.start()
        sR1.start()
        sL1.start()

        xf = x_ref[...]
        scores = jnp.dot(xf, rw_ref[...], preferred_element_type=jnp.float32)
        s_max = jnp.max(scores, axis=-1, keepdims=True)
        p = jnp.exp(scores - s_max)
        p = p / jnp.sum(p, axis=-1, keepdims=True)
        idx0 = idx_ref[:, 0:1]
        idx1 = idx_ref[:, 1:2]
        eids = lax.broadcasted_iota(jnp.int32, (n_tok, n_exp), 1)
        g0 = jnp.sum(jnp.where(eids == idx0, p, 0.0), axis=-1, keepdims=True)
        g1 = jnp.sum(jnp.where(eids == idx1, p, 0.0), axis=-1, keepdims=True)
        gs = g0 + g1
        g0 = g0 / gs
        g1 = g1 / gs

        def gate_for(e):
            return jnp.where(idx0 == e, g0, 0.0) + jnp.where(idx1 == e, g1, 0.0)

        def build_X(origins):
            xs_list = []
            for origin in origins:
                for k in range(e_loc):
                    xs_list.append(
                        (xf * gate_for(origin * e_loc + k)).astype(jnp.bfloat16)
                    )
            return jnp.concatenate(xs_list, axis=1) if len(xs_list) > 1 else xs_list[0]

        def w_mat(refs):
            ws = [r[...].reshape(e_loc * d, h_dim) for r in refs]
            return jnp.concatenate(ws, axis=0) if len(ws) > 1 else ws[0]

        acc = jnp.dot(build_X([my]), w_mat([my_w]),
                      preferred_element_type=jnp.float32)
        X_LR = build_X([left, right])
        X_diag = build_X([diag])

        copy(my_w.at[lo], w_fL.at[lo], 0, 0, left).wait_recv()
        fwd_R = copy(w_fL.at[lo], w_diag.at[lo], 4, 4, right)
        fwd_R.start()
        copy(my_w.at[hi], w_fR.at[hi], 2, 2, right).wait_recv()
        fwd_L = copy(w_fR.at[hi], w_diag.at[hi], 5, 5, left)
        fwd_L.start()

        copy(my_w.at[hi], w_fL.at[hi], 1, 1, left).wait_recv()
        copy(my_w.at[lo], w_fR.at[lo], 3, 3, right).wait_recv()
        acc = acc + jnp.dot(X_LR, w_mat([w_fL, w_fR]),
                            preferred_element_type=jnp.float32)

        copy(my_w.at[lo], w_diag.at[lo], 4, 4, left).wait_recv()
        copy(my_w.at[hi], w_diag.at[hi], 5, 5, right).wait_recv()
        out_ref[...] = (
            acc + jnp.dot(X_diag, w_mat([w_diag]),
                          preferred_element_type=jnp.float32)
        ).astype(jnp.bfloat16)

        for s in (sR0, sL0, sR1, sL1, fwd_R, fwd_L):
            s.wait_send()

    out_shape = jax.ShapeDtypeStruct((n_tok, h_dim), jnp.bfloat16)
    return pl.pallas_call(
        body,
        out_shape=out_shape,
        in_specs=[pl.BlockSpec(memory_space=pltpu.VMEM)] * 4,
        out_specs=pl.BlockSpec(memory_space=pltpu.VMEM),
        scratch_shapes=[
            pltpu.VMEM((e_loc, d, h_dim), jnp.bfloat16),
            pltpu.VMEM((e_loc, d, h_dim), jnp.bfloat16),
            pltpu.VMEM((e_loc, d, h_dim), jnp.bfloat16),
            pltpu.VMEM((e_loc, d, h_dim), jnp.bfloat16),
            pltpu.SemaphoreType.DMA((6,)),
            pltpu.SemaphoreType.DMA((6,)),
        ],
        compiler_params=pltpu.CompilerParams(collective_id=0),
    )(x, router_W, route_idx, expert_W)
